# baseline (speedup 1.0000x reference)
import jax
import jax.numpy as jnp
from jax.experimental import pallas as pl

_LBLK = 16384


def _body(xt_ref, emb_ref, w8_ref, o_ref):
    x0 = xt_ref[0:1, :]
    x1 = xt_ref[1:2, :]
    e0 = emb_ref[0, 0]
    e1 = emb_ref[0, 1]
    e = jnp.where(x1 >= 1.0, e1, e0)
    one = jnp.ones_like(x0)
    zero = jnp.zeros((5, x0.shape[1]), jnp.float32)
    m = jnp.concatenate([x0, e, one, zero], axis=0)
    o_ref[...] = jax.lax.dot_general(
        w8_ref[...], m,
        dimension_numbers=(((1,), (0,)), ((), ())),
        preferred_element_type=jnp.float32,
    )


@jax.jit
def _run(xt, emb_row, w8):
    B = xt.shape[1]
    N = w8.shape[0]
    grid = (B // _LBLK,)
    return pl.pallas_call(
        _body,
        grid=grid,
        in_specs=[
            pl.BlockSpec((2, _LBLK), lambda i: (0, i)),
            pl.BlockSpec((1, 2), lambda i: (0, 0)),
            pl.BlockSpec((N, 8), lambda i: (0, 0)),
        ],
        out_specs=pl.BlockSpec((N, _LBLK), lambda i: (0, i)),
        out_shape=jax.ShapeDtypeStruct((N, B), jnp.float32),
    )(xt, emb_row, w8)


def kernel(x, emb16, fc1_w, fc1_b):
    N = fc1_w.shape[0]
    xt = x.T
    emb_row = emb16.reshape(1, 2)
    w8 = jnp.concatenate(
        [fc1_w, fc1_b.reshape(N, 1), jnp.zeros((N, 5), jnp.float32)], axis=1
    )
    out_t = _run(xt, emb_row, w8)
    return out_t.T


# speedup vs baseline: 1.0801x; 1.0801x over previous
import jax
import jax.numpy as jnp
from jax.experimental import pallas as pl
from jax.experimental.pallas import tpu as pltpu

_NQ = 8


def _body(xt_ref, emb_ref, w8_ref, o_hbm, scratch, sems):
    B = xt_ref.shape[1]
    L = B // _NQ
    w8 = w8_ref[...]
    e0 = emb_ref[0, 0]
    e1 = emb_ref[0, 1]
    for q in range(_NQ):
        x0 = xt_ref[0:1, pl.ds(q * L, L)]
        x1 = xt_ref[1:2, pl.ds(q * L, L)]
        e = jnp.where(x1 >= 1.0, e1, e0)
        one = jnp.ones_like(x0)
        zero = jnp.zeros((5, L), jnp.float32)
        m = jnp.concatenate([x0, e, one, zero], axis=0)
        scratch[:, q * L:(q + 1) * L] = jax.lax.dot_general(
            w8, m,
            dimension_numbers=(((1,), (0,)), ((), ())),
            preferred_element_type=jnp.float32,
        )
        pltpu.make_async_copy(
            scratch.at[:, pl.ds(q * L, L)],
            o_hbm.at[:, pl.ds(q * L, L)],
            sems.at[q],
        ).start()
    for q in range(_NQ):
        pltpu.make_async_copy(
            scratch.at[:, pl.ds(q * L, L)],
            o_hbm.at[:, pl.ds(q * L, L)],
            sems.at[q],
        ).wait()


@jax.jit
def _run(xt, emb_row, w8):
    B = xt.shape[1]
    N = w8.shape[0]
    return pl.pallas_call(
        _body,
        in_specs=[
            pl.BlockSpec(memory_space=pltpu.MemorySpace.VMEM),
            pl.BlockSpec(memory_space=pltpu.MemorySpace.VMEM),
            pl.BlockSpec(memory_space=pltpu.MemorySpace.VMEM),
        ],
        out_specs=pl.BlockSpec(memory_space=pltpu.MemorySpace.HBM),
        out_shape=jax.ShapeDtypeStruct((N, B), jnp.float32),
        scratch_shapes=[
            pltpu.VMEM((N, B), jnp.float32),
            pltpu.SemaphoreType.DMA((_NQ,)),
        ],
    )(xt, emb_row, w8)


def kernel(x, emb16, fc1_w, fc1_b):
    N = fc1_w.shape[0]
    xt = x.T
    emb_row = emb16.reshape(1, 2)
    w8 = jnp.concatenate(
        [fc1_w, fc1_b.reshape(N, 1), jnp.zeros((N, 5), jnp.float32)], axis=1
    )
    out_t = _run(xt, emb_row, w8)
    return out_t.T
